# baseline (speedup 1.0000x reference)
import jax
import jax.numpy as jnp
from jax.experimental import pallas as pl

P = 0.15

_BLOCK_ROWS = 2000


def _mask_body(mask_ref, x_ref, o_ref):
    keep = (mask_ref[...] >= P).astype(x_ref.dtype)
    o_ref[...] = x_ref[...] * keep


def kernel(x, y, mask_u):
    n, d = x.shape
    grid = n // _BLOCK_ROWS
    x_masked = pl.pallas_call(
        _mask_body,
        grid=(grid,),
        in_specs=[
            pl.BlockSpec((1, d), lambda i: (0, 0)),
            pl.BlockSpec((_BLOCK_ROWS, d), lambda i: (i, 0)),
        ],
        out_specs=pl.BlockSpec((_BLOCK_ROWS, d), lambda i: (i, 0)),
        out_shape=jax.ShapeDtypeStruct((n, d), x.dtype),
    )(mask_u.reshape(1, d), x)
    return (x_masked, y)


# speedup vs baseline: 1.5407x; 1.5407x over previous
import jax
import jax.numpy as jnp
from jax.experimental import pallas as pl

P = 0.15

_BLOCK_ROWS = 10000


def _mask_body(mask_ref, x_ref, o_ref):
    keep = (mask_ref[...] >= P).astype(x_ref.dtype)
    o_ref[...] = x_ref[...] * keep


def kernel(x, y, mask_u):
    n, d = x.shape
    grid = n // _BLOCK_ROWS
    x_masked = pl.pallas_call(
        _mask_body,
        grid=(grid,),
        in_specs=[
            pl.BlockSpec((1, d), lambda i: (0, 0)),
            pl.BlockSpec((_BLOCK_ROWS, d), lambda i: (i, 0)),
        ],
        out_specs=pl.BlockSpec((_BLOCK_ROWS, d), lambda i: (i, 0)),
        out_shape=jax.ShapeDtypeStruct((n, d), x.dtype),
    )(mask_u.reshape(1, d), x)
    return (x_masked, y)


# speedup vs baseline: 1.6041x; 1.0411x over previous
import jax
import jax.numpy as jnp
from jax.experimental import pallas as pl

P = 0.15

_BLOCK_ROWS = 20000


def _mask_body(mask_ref, x_ref, o_ref):
    keep = (mask_ref[...] >= P).astype(x_ref.dtype)
    o_ref[...] = x_ref[...] * keep


def kernel(x, y, mask_u):
    n, d = x.shape
    grid = n // _BLOCK_ROWS
    x_masked = pl.pallas_call(
        _mask_body,
        grid=(grid,),
        in_specs=[
            pl.BlockSpec((1, d), lambda i: (0, 0)),
            pl.BlockSpec((_BLOCK_ROWS, d), lambda i: (i, 0)),
        ],
        out_specs=pl.BlockSpec((_BLOCK_ROWS, d), lambda i: (i, 0)),
        out_shape=jax.ShapeDtypeStruct((n, d), x.dtype),
    )(mask_u.reshape(1, d), x)
    return (x_masked, y)


# speedup vs baseline: 1.6122x; 1.0050x over previous
import jax
import jax.numpy as jnp
from jax.experimental import pallas as pl

P = 0.15

_BLOCK_ROWS = 25000


def _mask_body(mask_ref, x_ref, o_ref):
    keep = (mask_ref[...] >= P).astype(x_ref.dtype)
    o_ref[...] = x_ref[...] * keep


def kernel(x, y, mask_u):
    n, d = x.shape
    grid = n // _BLOCK_ROWS
    x_masked = pl.pallas_call(
        _mask_body,
        grid=(grid,),
        in_specs=[
            pl.BlockSpec((1, d), lambda i: (0, 0)),
            pl.BlockSpec((_BLOCK_ROWS, d), lambda i: (i, 0)),
        ],
        out_specs=pl.BlockSpec((_BLOCK_ROWS, d), lambda i: (i, 0)),
        out_shape=jax.ShapeDtypeStruct((n, d), x.dtype),
    )(mask_u.reshape(1, d), x)
    return (x_masked, y)
